# baseline (speedup 1.0000x reference)
import functools

import jax
import jax.numpy as jnp
from jax import lax
from jax.experimental import pallas as pl
from jax.experimental.pallas import tpu as pltpu
from jax.experimental.pallas import tpu_sc as plsc

_B, _S, _D = 4, 4096, 768
_N = _B * _S
_NC, _NS = 2, 16
_NW = _NC * _NS
_SPW = _S // _NW
_CH = 32
_NSUB = _SPW // _CH
_NCHUNK = _NSUB * _B
_NBUF = 4
_LAG = 2
_LANES = 16
_JV = _D // _LANES


def _make_emb_kernel():
    mesh = plsc.VectorSubcoreMesh(core_axis_name="c", subcore_axis_name="s")

    @functools.partial(
        pl.kernel,
        mesh=mesh,
        out_type=jax.ShapeDtypeStruct((_N, _D), jnp.float32),
        scratch_types=[
            pltpu.VMEM((_B, _SPW), jnp.int32),
            pltpu.VMEM((_NBUF, _CH, _D), jnp.float32),
            pltpu.VMEM((_CH, _D), jnp.float32),
            pltpu.SemaphoreType.DMA,
            pltpu.SemaphoreType.DMA((_NBUF,)),
            pltpu.SemaphoreType.DMA((_NBUF,)),
        ],
    )
    def emb(x_hbm, table_hbm, pe_hbm, out_hbm,
            idx_v, rows_v, pe_v, sem_i, sem_g, sem_o):
        wid = lax.axis_index("s") * _NC + lax.axis_index("c")
        s_base = wid * _SPW

        def gather_desc(t):
            sub, b = t // _B, t % _B
            idx_sl = idx_v.at[b, pl.ds(sub * _CH, _CH)]
            return pltpu.make_async_copy(table_hbm.at[idx_sl],
                                         rows_v.at[t % _NBUF],
                                         sem_g.at[t % _NBUF])

        def out_desc(t):
            sub, b = t // _B, t % _B
            row0 = b * _S + s_base + sub * _CH
            return pltpu.make_async_copy(rows_v.at[t % _NBUF],
                                         out_hbm.at[pl.ds(row0, _CH)],
                                         sem_o.at[t % _NBUF])

        pltpu.sync_copy(x_hbm.at[:, pl.ds(s_base, _SPW)], idx_v)

        @pl.loop(0, _NCHUNK + _LAG)
        def _it(i):
            @pl.when(i < _NCHUNK)
            def _():
                @pl.when(i >= _NBUF)
                def _():
                    out_desc(lax.max(i - _NBUF, 0)).wait()

                gather_desc(i).start()

            @pl.when(i >= _LAG)
            def _():
                t = lax.max(i - _LAG, 0)
                sub = t // _B

                @pl.when(t % _B == 0)
                def _():
                    pltpu.sync_copy(
                        pe_hbm.at[pl.ds(s_base + sub * _CH, _CH)], pe_v)

                gather_desc(t).wait()
                slot = t % _NBUF

                def row_body(r, carry):
                    for j in range(_JV):
                        sl = pl.ds(j * _LANES, _LANES)
                        plsc.addupdate(rows_v.at[slot, r, sl], pe_v[r, sl])
                    return carry

                lax.fori_loop(0, _CH, row_body, 0)
                out_desc(t).start()

        @pl.loop(_NCHUNK - _NBUF, _NCHUNK)
        def _drain(t):
            out_desc(t).wait()

    return emb


_emb = _make_emb_kernel()


def kernel(x, table, pe):
    out = _emb(x.astype(jnp.int32), table, pe)
    return out.reshape(_B, _S, _D)


# speedup vs baseline: 1.5225x; 1.5225x over previous
import functools

import jax
import jax.numpy as jnp
from jax import lax
from jax.experimental import pallas as pl
from jax.experimental.pallas import tpu as pltpu
from jax.experimental.pallas import tpu_sc as plsc

_B, _S, _D = 4, 4096, 768
_N = _B * _S
_NC, _NS = 2, 16
_NW = _NC * _NS
_SPW = _S // _NW
_CH = 32
_NSUB = _SPW // _CH
_NCHUNK = _NSUB * _B
_LANES = 16
_JV = _D // _LANES


def _make_emb_kernel():
    mesh = plsc.VectorSubcoreMesh(core_axis_name="c", subcore_axis_name="s")

    @functools.partial(
        pl.kernel,
        mesh=mesh,
        out_type=jax.ShapeDtypeStruct((_N, _D), jnp.float32),
        scratch_types=[
            pltpu.VMEM((_B, _SPW), jnp.int32),
            pltpu.VMEM((2, _CH, _D), jnp.float32),
            pltpu.VMEM((_CH, _D), jnp.float32),
            pltpu.SemaphoreType.DMA,
            pltpu.SemaphoreType.DMA,
            pltpu.SemaphoreType.DMA,
            pltpu.SemaphoreType.DMA,
            pltpu.SemaphoreType.DMA,
            pltpu.SemaphoreType.DMA,
        ],
    )
    def emb(x_hbm, table_hbm, pe_hbm, out_hbm,
            idx_v, rows_v, pe_v, sem_i, sem_pe,
            sem_g0, sem_g1, sem_o0, sem_o1):
        wid = lax.axis_index("s") * _NC + lax.axis_index("c")
        s_base = wid * _SPW
        sems_g = (sem_g0, sem_g1)
        sems_o = (sem_o0, sem_o1)

        def coords(t):
            sub = t // _B
            b = t % _B
            row0 = b * _S + s_base + sub * _CH
            return sub, b, row0

        def pe_desc(sub):
            return pltpu.make_async_copy(
                pe_hbm.at[pl.ds(s_base + sub * _CH, _CH)], pe_v, sem_pe)

        idx_desc = pltpu.make_async_copy(
            x_hbm.at[:, pl.ds(s_base, _SPW)], idx_v, sem_i)
        idx_desc.start()
        pe_desc(0).start()
        idx_desc.wait()

        @pl.loop(0, _NCHUNK, step=2)
        def _chunks(c):
            for k in range(2):
                t = c + k
                sub, b, row0 = coords(t)

                @pl.when(t % _B == 0)
                def _():
                    pe_desc(sub).wait()

                @pl.when(c > 0)
                def _():
                    tp = lax.max(t - 2, 0)
                    _, _, row0p = coords(tp)
                    pltpu.make_async_copy(
                        rows_v.at[k], out_hbm.at[pl.ds(row0p, _CH)],
                        sems_o[k]).wait()

                idx_sl = idx_v.at[b, pl.ds(sub * _CH, _CH)]
                pltpu.async_copy(table_hbm.at[idx_sl], rows_v.at[k],
                                 sems_g[k])

            for k in range(2):
                t = c + k
                sub, b, row0 = coords(t)
                idx_sl = idx_v.at[b, pl.ds(sub * _CH, _CH)]
                pltpu.make_async_copy(table_hbm.at[idx_sl], rows_v.at[k],
                                      sems_g[k]).wait()

                def row_body(r, carry):
                    for j in range(_JV):
                        sl = pl.ds(j * _LANES, _LANES)
                        plsc.addupdate(rows_v.at[k, r, sl], pe_v[r, sl])
                    return carry

                lax.fori_loop(0, _CH, row_body, 0)
                pltpu.async_copy(rows_v.at[k], out_hbm.at[pl.ds(row0, _CH)],
                                 sems_o[k])

                @pl.when((t % _B == _B - 1) & (t + 1 < _NCHUNK))
                def _():
                    pe_desc(lax.min(sub + 1, _NSUB - 1)).start()

        for k in range(2):
            t = _NCHUNK - 2 + k
            _, _, row0 = coords(t)
            pltpu.make_async_copy(rows_v.at[k], out_hbm.at[pl.ds(row0, _CH)],
                                  sems_o[k]).wait()

    return emb


_emb = _make_emb_kernel()


def kernel(x, table, pe):
    out = _emb(x.astype(jnp.int32), table, pe)
    return out.reshape(_B, _S, _D)
